# baseline (speedup 1.0000x reference)
import functools

import jax
import jax.numpy as jnp
from jax import lax
from jax.experimental import pallas as pl
from jax.experimental.pallas import tpu as pltpu
from jax.experimental.pallas import tpu_sc as plsc

N = 10000
E = 320000
D_IN = 128
H1 = 16
H2 = 10

NC = 2
NS = 16
NW = NC * NS
B = 128
E_PAD = 327680
NCHUNKS = E_PAD // B
CW = NCHUNKS // NW
N_ACC = 10240
RPT = N_ACC // NS

_SC_MESH = plsc.VectorSubcoreMesh(core_axis_name="c", subcore_axis_name="s")
_SC_PARAMS = pltpu.CompilerParams(use_tc_tiling_on_sc=False)


def _deg_body(dst_hbm, ones_hbm, zeros_hbm, out_hbm, idx_d, rows, acc_sh, sem):
    c = lax.axis_index("c")
    s = lax.axis_index("s")
    w = c * NS + s
    pltpu.sync_copy(zeros_hbm.at[pl.ds(s * RPT, RPT)],
                    acc_sh.at[pl.ds(s * RPT, RPT)])
    pltpu.sync_copy(ones_hbm, rows)
    pltpu.sync_copy(dst_hbm.at[pl.ds(w * CW, CW)], idx_d)
    plsc.subcore_barrier()

    def fire(j, carry):
        pltpu.async_copy(rows, acc_sh.at[idx_d.at[j]], sem, add=True)
        return carry

    lax.fori_loop(0, CW, fire, 0)

    def drain(j, carry):
        pltpu.make_async_copy(rows, acc_sh.at[idx_d.at[j]], sem).wait()
        return carry

    lax.fori_loop(0, CW, drain, 0)
    plsc.subcore_barrier()
    pltpu.sync_copy(acc_sh.at[pl.ds(s * RPT, RPT)],
                    out_hbm.at[c, pl.ds(s * RPT, RPT)])


_deg_kernel = pl.kernel(
    _deg_body,
    out_type=jax.ShapeDtypeStruct((NC, N_ACC, H1), jnp.float32),
    mesh=_SC_MESH,
    scratch_types=[
        pltpu.VMEM((CW, B), jnp.int32),
        pltpu.VMEM((B, H1), jnp.float32),
        pltpu.VMEM_SHARED((N_ACC, H1), jnp.float32),
        pltpu.SemaphoreType.DMA,
    ],
    compiler_params=_SC_PARAMS,
)


NBUF = 8
NGRP = CW // NBUF


def _segsum_body(table_hbm, src_hbm, dst_hbm, zeros_hbm, out_hbm,
                 idx_s, idx_d, rows, acc_sh, gsem, ssem):
    c = lax.axis_index("c")
    s = lax.axis_index("s")
    w = c * NS + s
    pltpu.sync_copy(zeros_hbm.at[pl.ds(s * RPT, RPT)],
                    acc_sh.at[pl.ds(s * RPT, RPT)])
    pltpu.sync_copy(src_hbm.at[pl.ds(w * CW, CW)], idx_s)
    pltpu.sync_copy(dst_hbm.at[pl.ds(w * CW, CW)], idx_d)
    plsc.subcore_barrier()

    for b in range(NBUF):
        pltpu.async_copy(table_hbm.at[idx_s.at[b]], rows.at[0, b], gsem)

    def group(g, carry):
        gm = g % 2
        for b in range(NBUF):
            pltpu.make_async_copy(table_hbm.at[idx_s.at[g * NBUF + b]],
                                  rows.at[gm, b], gsem).wait()

        @pl.when(g >= 1)
        def _drain_prev():
            for b in range(NBUF):
                pltpu.make_async_copy(
                    rows.at[1 - gm, b],
                    acc_sh.at[idx_d.at[(g - 1) * NBUF + b]], ssem).wait()

        @pl.when(g + 1 < NGRP)
        def _fire_next():
            for b in range(NBUF):
                pltpu.async_copy(table_hbm.at[idx_s.at[(g + 1) * NBUF + b]],
                                 rows.at[1 - gm, b], gsem)

        for b in range(NBUF):
            pltpu.async_copy(rows.at[gm, b],
                             acc_sh.at[idx_d.at[g * NBUF + b]], ssem,
                             add=True)
        return carry

    lax.fori_loop(0, NGRP, group, 0)
    for b in range(NBUF):
        pltpu.make_async_copy(rows.at[(NGRP - 1) % 2, b],
                              acc_sh.at[idx_d.at[(NGRP - 1) * NBUF + b]],
                              ssem).wait()
    plsc.subcore_barrier()
    pltpu.sync_copy(acc_sh.at[pl.ds(s * RPT, RPT)],
                    out_hbm.at[c, pl.ds(s * RPT, RPT)])


_segsum_kernel = pl.kernel(
    _segsum_body,
    out_type=jax.ShapeDtypeStruct((NC, N_ACC, H1), jnp.float32),
    mesh=_SC_MESH,
    scratch_types=[
        pltpu.VMEM((CW, B), jnp.int32),
        pltpu.VMEM((CW, B), jnp.int32),
        pltpu.VMEM((2, NBUF, B, H1), jnp.float32),
        pltpu.VMEM_SHARED((N_ACC, H1), jnp.float32),
        pltpu.SemaphoreType.DMA,
        pltpu.SemaphoreType.DMA,
    ],
    compiler_params=_SC_PARAMS,
)

NP = N // 8
NP_ACC = N_ACC // 8


def _dinv_packed(degp):
    return lax.rsqrt(degp[0, :NP] + degp[1, :NP] + 1.0)


def _pre_a_body(xgp, w1k, xlp, wl1k, bl1p, h1r_p, xloc_p):
    h1r_p[...] = jnp.dot(xgp[...], w1k[...],
                         preferred_element_type=jnp.float32)
    xloc_p[...] = (
        jnp.dot(xlp[...], wl1k[...], preferred_element_type=jnp.float32)
        + bl1p[...]
    )


def _pre_b_body(degp, h1r, h1p_p):
    h1p_p[...] = h1r[...] * _dinv_packed(degp)


def _mid_body(degp, acc, h1p, b1p, w2k, h2p_p):
    dinv = _dinv_packed(degp)
    x1_p = jnp.maximum(
        dinv * (acc[0, :NP] + acc[1, :NP] + h1p[...]) + b1p[...], 0.0)
    h2 = jnp.dot(x1_p, w2k[...], preferred_element_type=jnp.float32)
    h2p_p[...] = h2 * dinv


def _post_body(degp, acc, h2p, b2p, xloc, wl2ak, wl2bk, bl2p, wl3k, bl3p,
               sel_a, sel_b, sel_at, sel_bt, out_p):
    dinv = _dinv_packed(degp)
    s2_p = dinv * (acc[0, :NP] + acc[1, :NP] + h2p[...]) + b2p[...]
    t = (jnp.dot(jnp.maximum(s2_p, 0.0), wl2ak[...],
                 preferred_element_type=jnp.float32)
         + jnp.dot(jnp.maximum(xloc[...], 0.0), wl2bk[...],
                   preferred_element_type=jnp.float32)
         + bl2p[...])
    y = jnp.maximum(t, 0.0)
    z = jnp.dot(y, wl3k[...], preferred_element_type=jnp.float32) + bl3p[...]
    za = jnp.dot(z, sel_a[...], preferred_element_type=jnp.float32)
    zb = jnp.dot(z, sel_b[...], preferred_element_type=jnp.float32)
    m = jnp.maximum(za, zb)
    lse = m + jnp.log(jnp.exp(za - m) + jnp.exp(zb - m))
    oa = za - lse
    ob = zb - lse
    out_p[...] = (
        jnp.dot(oa, sel_at[...], preferred_element_type=jnp.float32)
        + jnp.dot(ob, sel_bt[...], preferred_element_type=jnp.float32))


def _full(shape):
    nd = len(shape)
    return pl.BlockSpec(shape, lambda: (0,) * nd)


_pre_a_call = pl.pallas_call(
    _pre_a_body,
    in_specs=[_full((NP, 1024)), _full((1024, 128)), _full((NP, 752)),
              _full((752, 800)), _full((1, 800))],
    out_specs=[_full((NP, 128)), _full((NP, 800))],
    out_shape=[jax.ShapeDtypeStruct((NP, 128), jnp.float32),
               jax.ShapeDtypeStruct((NP, 800), jnp.float32)],
)

_pre_b_call = pl.pallas_call(
    _pre_b_body,
    in_specs=[_full((NC, NP_ACC, 128)), _full((NP, 128))],
    out_specs=[_full((NP, 128))],
    out_shape=[jax.ShapeDtypeStruct((NP, 128), jnp.float32)],
)

_mid_call = pl.pallas_call(
    _mid_body,
    in_specs=[_full((NC, NP_ACC, 128)), _full((NC, NP_ACC, 128)),
              _full((NP, 128)), _full((1, 128)), _full((128, 128))],
    out_specs=[_full((NP, 128))],
    out_shape=[jax.ShapeDtypeStruct((NP, 128), jnp.float32)],
)

_post_call = pl.pallas_call(
    _post_body,
    in_specs=[_full((NC, NP_ACC, 128)), _full((NC, NP_ACC, 128)),
              _full((NP, 128)), _full((1, 128)), _full((NP, 800)),
              _full((128, 648)), _full((800, 648)), _full((1, 648)),
              _full((648, 16)), _full((1, 16)), _full((16, 8)),
              _full((16, 8)), _full((8, 16)), _full((8, 16))],
    out_specs=[_full((NP, 16))],
    out_shape=[jax.ShapeDtypeStruct((NP, 16), jnp.float32)],
)


def kernel(x_graph, x_linear, edge_index, W1, b1, W2, b2,
           Wl1, bl1, Wl2, bl2, Wl3, bl3):
    f32 = jnp.float32
    ei = edge_index.astype(jnp.int32)
    pad_n = E_PAD - E
    src2d = jnp.concatenate(
        [ei[0], jnp.zeros((pad_n,), jnp.int32)]).reshape(NCHUNKS, B)
    dst2d = jnp.concatenate(
        [ei[1], jnp.full((pad_n,), N_ACC - 1, jnp.int32)]).reshape(NCHUNKS, B)
    zeros = jnp.zeros((N_ACC, H1), f32)
    ones_blk = jnp.ones((B, H1), f32)

    eye8 = jnp.eye(8, dtype=f32)
    w2p = jnp.pad(W2, ((0, 0), (0, H1 - H2)))
    wl2a = jnp.pad(Wl2[:H2], ((0, H1 - H2), (0, 0)))
    wl2b = Wl2[H2:]
    w1k = jnp.kron(eye8, W1)
    wl1k = jnp.kron(eye8, Wl1)
    w2k = jnp.kron(eye8, w2p)
    wl2ak = jnp.kron(eye8, wl2a)
    wl2bk = jnp.kron(eye8, wl2b)
    wl3k = jnp.kron(eye8, Wl3)
    sel_a = jnp.kron(eye8, jnp.array([[1.0], [0.0]], f32))
    sel_b = jnp.kron(eye8, jnp.array([[0.0], [1.0]], f32))
    b1p = jnp.tile(b1, 8).reshape(1, 128)
    b2p = jnp.tile(jnp.pad(b2, (0, H1 - H2)), 8).reshape(1, 128)
    bl1p = jnp.tile(bl1, 8).reshape(1, 800)
    bl2p = jnp.tile(bl2, 8).reshape(1, 648)
    bl3p = jnp.tile(bl3, 8).reshape(1, 16)

    xgp = x_graph.reshape(NP, 1024)
    xlp = x_linear.reshape(NP, 752)

    h1r_p, xloc_p = _pre_a_call(xgp, w1k, xlp, wl1k, bl1p)
    degp = _deg_kernel(dst2d, ones_blk, zeros)
    degp_p = degp.reshape(NC, NP_ACC, 128)
    (h1p_p,) = _pre_b_call(degp_p, h1r_p)
    acc1 = _segsum_kernel(h1p_p.reshape(N, H1), src2d, dst2d, zeros)
    (h2p_p,) = _mid_call(degp_p, acc1.reshape(NC, NP_ACC, 128), h1p_p,
                         b1p, w2k)
    acc2 = _segsum_kernel(h2p_p.reshape(N, H1), src2d, dst2d, zeros)
    (out_p,) = _post_call(degp_p, acc2.reshape(NC, NP_ACC, 128), h2p_p,
                          b2p, xloc_p, wl2ak, wl2bk, bl2p, wl3k, bl3p,
                          sel_a, sel_b, sel_a.T, sel_b.T)
    return out_p.reshape(N, 2)


# speedup vs baseline: 1.0053x; 1.0053x over previous
import functools

import jax
import jax.numpy as jnp
from jax import lax
from jax.experimental import pallas as pl
from jax.experimental.pallas import tpu as pltpu
from jax.experimental.pallas import tpu_sc as plsc

N = 10000
E = 320000
D_IN = 128
H1 = 16
H2 = 10

NC = 2
NS = 16
NW = NC * NS
B = 128
E_PAD = 327680
NCHUNKS = E_PAD // B
CW = NCHUNKS // NW
N_ACC = 10240
RPT = N_ACC // NS

_SC_MESH = plsc.VectorSubcoreMesh(core_axis_name="c", subcore_axis_name="s")
_SC_PARAMS = pltpu.CompilerParams(use_tc_tiling_on_sc=False)


def _deg_body(dst_hbm, ones_hbm, zeros_hbm, out_hbm, idx_d, rows, acc_sh, sem):
    c = lax.axis_index("c")
    s = lax.axis_index("s")
    w = c * NS + s
    pltpu.sync_copy(zeros_hbm.at[pl.ds(s * RPT, RPT)],
                    acc_sh.at[pl.ds(s * RPT, RPT)])
    pltpu.sync_copy(ones_hbm, rows)
    pltpu.sync_copy(dst_hbm.at[pl.ds(w * CW, CW)], idx_d)
    plsc.subcore_barrier()

    def fire(j, carry):
        pltpu.async_copy(rows, acc_sh.at[idx_d.at[j]], sem, add=True)
        return carry

    lax.fori_loop(0, CW, fire, 0)

    def drain(j, carry):
        pltpu.make_async_copy(rows, acc_sh.at[idx_d.at[j]], sem).wait()
        return carry

    lax.fori_loop(0, CW, drain, 0)
    plsc.subcore_barrier()
    pltpu.sync_copy(acc_sh.at[pl.ds(s * RPT, RPT)],
                    out_hbm.at[c, pl.ds(s * RPT, RPT)])


_deg_kernel = pl.kernel(
    _deg_body,
    out_type=jax.ShapeDtypeStruct((NC, N_ACC, H1), jnp.float32),
    mesh=_SC_MESH,
    scratch_types=[
        pltpu.VMEM((CW, B), jnp.int32),
        pltpu.VMEM((B, H1), jnp.float32),
        pltpu.VMEM_SHARED((N_ACC, H1), jnp.float32),
        pltpu.SemaphoreType.DMA,
    ],
    compiler_params=_SC_PARAMS,
)


NBUF = 8
NGRP = CW // NBUF


def _segsum_body(table_hbm, src_hbm, dst_hbm, zeros_hbm, out_hbm,
                 idx_s, idx_d, rows, acc_sh, gsem, ssem):
    c = lax.axis_index("c")
    s = lax.axis_index("s")
    w = c * NS + s
    pltpu.sync_copy(zeros_hbm.at[pl.ds(s * RPT, RPT)],
                    acc_sh.at[pl.ds(s * RPT, RPT)])
    pltpu.sync_copy(src_hbm.at[pl.ds(w * CW, CW)], idx_s)
    pltpu.sync_copy(dst_hbm.at[pl.ds(w * CW, CW)], idx_d)
    plsc.subcore_barrier()

    for b in range(NBUF):
        pltpu.async_copy(table_hbm.at[idx_s.at[b]], rows.at[0, b], gsem)

    def group(g, carry):
        gm = g % 2
        for b in range(NBUF):
            pltpu.make_async_copy(table_hbm.at[idx_s.at[g * NBUF + b]],
                                  rows.at[gm, b], gsem).wait()

        @pl.when(g >= 1)
        def _drain_prev():
            for b in range(NBUF):
                pltpu.make_async_copy(
                    rows.at[1 - gm, b],
                    acc_sh.at[idx_d.at[(g - 1) * NBUF + b]], ssem).wait()

        @pl.when(g + 1 < NGRP)
        def _fire_next():
            for b in range(NBUF):
                pltpu.async_copy(table_hbm.at[idx_s.at[(g + 1) * NBUF + b]],
                                 rows.at[1 - gm, b], gsem)

        for b in range(NBUF):
            pltpu.async_copy(rows.at[gm, b],
                             acc_sh.at[idx_d.at[g * NBUF + b]], ssem,
                             add=True)
        return carry

    lax.fori_loop(0, NGRP, group, 0)
    for b in range(NBUF):
        pltpu.make_async_copy(rows.at[(NGRP - 1) % 2, b],
                              acc_sh.at[idx_d.at[(NGRP - 1) * NBUF + b]],
                              ssem).wait()
    plsc.subcore_barrier()
    pltpu.sync_copy(acc_sh.at[pl.ds(s * RPT, RPT)],
                    out_hbm.at[c, pl.ds(s * RPT, RPT)])


_segsum_kernel = pl.kernel(
    _segsum_body,
    out_type=jax.ShapeDtypeStruct((NC, N_ACC, H1), jnp.float32),
    mesh=_SC_MESH,
    scratch_types=[
        pltpu.VMEM((CW, B), jnp.int32),
        pltpu.VMEM((CW, B), jnp.int32),
        pltpu.VMEM((2, NBUF, B, H1), jnp.float32),
        pltpu.VMEM_SHARED((N_ACC, H1), jnp.float32),
        pltpu.SemaphoreType.DMA,
        pltpu.SemaphoreType.DMA,
    ],
    compiler_params=_SC_PARAMS,
)

NP = N // 8
NP_ACC = N_ACC // 8


def _dinv_packed(degp):
    return lax.rsqrt(degp[0, :NP] + degp[1, :NP] + 1.0)


def _pre_a_body(xgp, w1k, xlp, wl1k, bl1p, h1r_p, xloc_p):
    h1r_p[...] = jnp.dot(xgp[...], w1k[...],
                         preferred_element_type=jnp.float32)
    xloc_p[...] = (
        jnp.dot(xlp[...], wl1k[...], preferred_element_type=jnp.float32)
        + bl1p[...]
    )


def _pre_b_body(degp, h1r, h1p_p):
    h1p_p[...] = h1r[...] * _dinv_packed(degp)


def _mid_body(degp, acc, h1p, b1p, w2k, h2p_p):
    dinv = _dinv_packed(degp)
    x1_p = jnp.maximum(
        dinv * (acc[0, :NP] + acc[1, :NP] + h1p[...]) + b1p[...], 0.0)
    h2 = jnp.dot(x1_p, w2k[...], preferred_element_type=jnp.float32)
    h2p_p[...] = h2 * dinv


def _post_body(degp, acc, h2p, b2p, xloc, wl2ak, wl2bk, bl2p, wl3k, bl3p,
               sel_a, sel_b, sel_at, sel_bt, out_p):
    dinv = _dinv_packed(degp)
    s2_p = dinv * (acc[0, :NP] + acc[1, :NP] + h2p[...]) + b2p[...]
    t = (jnp.dot(jnp.maximum(s2_p, 0.0), wl2ak[...],
                 preferred_element_type=jnp.float32)
         + jnp.dot(jnp.maximum(xloc[...], 0.0), wl2bk[...],
                   preferred_element_type=jnp.float32)
         + bl2p[...])
    y = jnp.maximum(t, 0.0)
    z = jnp.dot(y, wl3k[...], preferred_element_type=jnp.float32) + bl3p[...]
    za = jnp.dot(z, sel_a[...], preferred_element_type=jnp.float32)
    zb = jnp.dot(z, sel_b[...], preferred_element_type=jnp.float32)
    m = jnp.maximum(za, zb)
    lse = m + jnp.log(jnp.exp(za - m) + jnp.exp(zb - m))
    oa = za - lse
    ob = zb - lse
    out_p[...] = (
        jnp.dot(oa, sel_at[...], preferred_element_type=jnp.float32)
        + jnp.dot(ob, sel_bt[...], preferred_element_type=jnp.float32))


def _full(shape):
    nd = len(shape)
    return pl.BlockSpec(shape, lambda: (0,) * nd)


_pre_a_call = pl.pallas_call(
    _pre_a_body,
    in_specs=[_full((NP, 1024)), _full((1024, 128)), _full((NP, 752)),
              _full((752, 800)), _full((1, 800))],
    out_specs=[_full((NP, 128)), _full((NP, 800))],
    out_shape=[jax.ShapeDtypeStruct((NP, 128), jnp.float32),
               jax.ShapeDtypeStruct((NP, 800), jnp.float32)],
)

_pre_b_call = pl.pallas_call(
    _pre_b_body,
    in_specs=[_full((NC, NP_ACC, 128)), _full((NP, 128))],
    out_specs=[_full((NP, 128))],
    out_shape=[jax.ShapeDtypeStruct((NP, 128), jnp.float32)],
)

_mid_call = pl.pallas_call(
    _mid_body,
    in_specs=[_full((NC, NP_ACC, 128)), _full((NC, NP_ACC, 128)),
              _full((NP, 128)), _full((1, 128)), _full((128, 128))],
    out_specs=[_full((NP, 128))],
    out_shape=[jax.ShapeDtypeStruct((NP, 128), jnp.float32)],
)

_post_call = pl.pallas_call(
    _post_body,
    in_specs=[_full((NC, NP_ACC, 128)), _full((NC, NP_ACC, 128)),
              _full((NP, 128)), _full((1, 128)), _full((NP, 800)),
              _full((128, 648)), _full((800, 648)), _full((1, 648)),
              _full((648, 16)), _full((1, 16)), _full((16, 8)),
              _full((16, 8)), _full((8, 16)), _full((8, 16))],
    out_specs=[_full((NP, 16))],
    out_shape=[jax.ShapeDtypeStruct((NP, 16), jnp.float32)],
)


def kernel(x_graph, x_linear, edge_index, W1, b1, W2, b2,
           Wl1, bl1, Wl2, bl2, Wl3, bl3):
    f32 = jnp.float32
    ei = edge_index.astype(jnp.int32)
    pad_n = E_PAD - E
    src2d = jnp.concatenate(
        [ei[0], jnp.zeros((pad_n,), jnp.int32)]).reshape(NCHUNKS, B)
    pad_dst = N + jnp.arange(pad_n, dtype=jnp.int32) % (N_ACC - N)
    dst2d = jnp.concatenate([ei[1], pad_dst]).reshape(NCHUNKS, B)
    zeros = jnp.zeros((N_ACC, H1), f32)
    ones_blk = jnp.ones((B, H1), f32)

    eye8 = jnp.eye(8, dtype=f32)
    w2p = jnp.pad(W2, ((0, 0), (0, H1 - H2)))
    wl2a = jnp.pad(Wl2[:H2], ((0, H1 - H2), (0, 0)))
    wl2b = Wl2[H2:]
    w1k = jnp.kron(eye8, W1)
    wl1k = jnp.kron(eye8, Wl1)
    w2k = jnp.kron(eye8, w2p)
    wl2ak = jnp.kron(eye8, wl2a)
    wl2bk = jnp.kron(eye8, wl2b)
    wl3k = jnp.kron(eye8, Wl3)
    sel_a = jnp.kron(eye8, jnp.array([[1.0], [0.0]], f32))
    sel_b = jnp.kron(eye8, jnp.array([[0.0], [1.0]], f32))
    b1p = jnp.tile(b1, 8).reshape(1, 128)
    b2p = jnp.tile(jnp.pad(b2, (0, H1 - H2)), 8).reshape(1, 128)
    bl1p = jnp.tile(bl1, 8).reshape(1, 800)
    bl2p = jnp.tile(bl2, 8).reshape(1, 648)
    bl3p = jnp.tile(bl3, 8).reshape(1, 16)

    xgp = x_graph.reshape(NP, 1024)
    xlp = x_linear.reshape(NP, 752)

    h1r_p, xloc_p = _pre_a_call(xgp, w1k, xlp, wl1k, bl1p)
    degp = _deg_kernel(dst2d, ones_blk, zeros)
    degp_p = degp.reshape(NC, NP_ACC, 128)
    (h1p_p,) = _pre_b_call(degp_p, h1r_p)
    acc1 = _segsum_kernel(h1p_p.reshape(N, H1), src2d, dst2d, zeros)
    (h2p_p,) = _mid_call(degp_p, acc1.reshape(NC, NP_ACC, 128), h1p_p,
                         b1p, w2k)
    acc2 = _segsum_kernel(h2p_p.reshape(N, H1), src2d, dst2d, zeros)
    (out_p,) = _post_call(degp_p, acc2.reshape(NC, NP_ACC, 128), h2p_p,
                          b2p, xloc_p, wl2ak, wl2bk, bl2p, wl3k, bl3p,
                          sel_a, sel_b, sel_a.T, sel_b.T)
    return out_p.reshape(N, 2)


# speedup vs baseline: 1.6023x; 1.5938x over previous
import functools

import jax
import jax.numpy as jnp
from jax import lax
from jax.experimental import pallas as pl
from jax.experimental.pallas import tpu as pltpu
from jax.experimental.pallas import tpu_sc as plsc

N = 10000
E = 320000
D_IN = 128
H1 = 16
H2 = 10

NC = 2
NS = 16
NW = NC * NS
B = 128
E_PAD = 327680
NCHUNKS = E_PAD // B
CW = NCHUNKS // NW
N_ACC = 10240
RPT = N_ACC // NS

_SC_MESH = plsc.VectorSubcoreMesh(core_axis_name="c", subcore_axis_name="s")
_SC_PARAMS = pltpu.CompilerParams(use_tc_tiling_on_sc=False)


def _deg_body(dst_hbm, ones_hbm, zeros_hbm, out_hbm, idx_d, rows, acc_sh, sem):
    c = lax.axis_index("c")
    s = lax.axis_index("s")
    w = c * NS + s
    pltpu.sync_copy(zeros_hbm.at[pl.ds(s * RPT, RPT)],
                    acc_sh.at[pl.ds(s * RPT, RPT)])
    pltpu.sync_copy(ones_hbm, rows)
    pltpu.sync_copy(dst_hbm.at[pl.ds(w * CW, CW)], idx_d)
    plsc.subcore_barrier()

    def fire(j, carry):
        pltpu.async_copy(rows, acc_sh.at[idx_d.at[j]], sem, add=True)
        return carry

    lax.fori_loop(0, CW, fire, 0)

    def drain(j, carry):
        pltpu.make_async_copy(rows, acc_sh.at[idx_d.at[j]], sem).wait()
        return carry

    lax.fori_loop(0, CW, drain, 0)
    plsc.subcore_barrier()
    pltpu.sync_copy(acc_sh.at[pl.ds(s * RPT, RPT)],
                    out_hbm.at[c, pl.ds(s * RPT, RPT)])


_deg_kernel = pl.kernel(
    _deg_body,
    out_type=jax.ShapeDtypeStruct((NC, N_ACC, H1), jnp.float32),
    mesh=_SC_MESH,
    scratch_types=[
        pltpu.VMEM((CW, B), jnp.int32),
        pltpu.VMEM((B, H1), jnp.float32),
        pltpu.VMEM_SHARED((N_ACC, H1), jnp.float32),
        pltpu.SemaphoreType.DMA,
    ],
    compiler_params=_SC_PARAMS,
)


NBUF = 8
NGRP = CW // NBUF


def _segsum_body(table_hbm, src_hbm, dst_hbm, zeros_hbm, out_hbm,
                 idx_s, idx_d, rows, acc_sh, gsem, ssem):
    c = lax.axis_index("c")
    s = lax.axis_index("s")
    w = c * NS + s
    pltpu.sync_copy(zeros_hbm.at[pl.ds(s * RPT, RPT)],
                    acc_sh.at[pl.ds(s * RPT, RPT)])
    pltpu.sync_copy(src_hbm.at[pl.ds(w * CW, CW)], idx_s)
    pltpu.sync_copy(dst_hbm.at[pl.ds(w * CW, CW)], idx_d)
    plsc.subcore_barrier()

    for b in range(NBUF):
        pltpu.async_copy(table_hbm.at[idx_s.at[b]], rows.at[0, b], gsem)

    def group(g, carry):
        gm = g % 2
        for b in range(NBUF):
            pltpu.make_async_copy(table_hbm.at[idx_s.at[g * NBUF + b]],
                                  rows.at[gm, b], gsem).wait()

        @pl.when(g >= 1)
        def _drain_prev():
            for b in range(NBUF):
                pltpu.make_async_copy(
                    rows.at[1 - gm, b],
                    acc_sh.at[idx_d.at[(g - 1) * NBUF + b]], ssem).wait()

        @pl.when(g + 1 < NGRP)
        def _fire_next():
            for b in range(NBUF):
                pltpu.async_copy(table_hbm.at[idx_s.at[(g + 1) * NBUF + b]],
                                 rows.at[1 - gm, b], gsem)

        for b in range(NBUF):
            pltpu.async_copy(rows.at[gm, b],
                             acc_sh.at[idx_d.at[g * NBUF + b]], ssem,
                             add=True)
        return carry

    lax.fori_loop(0, NGRP, group, 0)
    for b in range(NBUF):
        pltpu.make_async_copy(rows.at[(NGRP - 1) % 2, b],
                              acc_sh.at[idx_d.at[(NGRP - 1) * NBUF + b]],
                              ssem).wait()
    plsc.subcore_barrier()
    pltpu.sync_copy(acc_sh.at[pl.ds(s * RPT, RPT)],
                    out_hbm.at[c, pl.ds(s * RPT, RPT)])


_segsum_kernel = pl.kernel(
    _segsum_body,
    out_type=jax.ShapeDtypeStruct((NC, N_ACC, H1), jnp.float32),
    mesh=_SC_MESH,
    scratch_types=[
        pltpu.VMEM((CW, B), jnp.int32),
        pltpu.VMEM((CW, B), jnp.int32),
        pltpu.VMEM((2, NBUF, B, H1), jnp.float32),
        pltpu.VMEM_SHARED((N_ACC, H1), jnp.float32),
        pltpu.SemaphoreType.DMA,
        pltpu.SemaphoreType.DMA,
    ],
    compiler_params=_SC_PARAMS,
)

NP = N // 8
NP_ACC = N_ACC // 8


def _dinv_packed(degp):
    return lax.rsqrt(degp[0, :NP] + degp[1, :NP] + 1.0)


def _pre_a_body(xgp, w1k, xlp, wl1k, bl1p, h1r_p, xloc_p):
    h1r_p[...] = jnp.dot(xgp[...], w1k[...],
                         preferred_element_type=jnp.float32)
    xloc_p[...] = (
        jnp.dot(xlp[...], wl1k[...], preferred_element_type=jnp.float32)
        + bl1p[...]
    )


def _pre_b_body(degp, h1r, h1p_p):
    h1p_p[...] = h1r[...] * _dinv_packed(degp)


def _mid_body(degp, acc, h1p, b1p, w2k, h2p_p):
    dinv = _dinv_packed(degp)
    x1_p = jnp.maximum(
        dinv * (acc[0, :NP] + acc[1, :NP] + h1p[...]) + b1p[...], 0.0)
    h2 = jnp.dot(x1_p, w2k[...], preferred_element_type=jnp.float32)
    h2p_p[...] = h2 * dinv


def _post_body(degp, acc, h2p, b2p, xloc, wl2ak, wl2bk, bl2p, wl3k, bl3p,
               sel_a, sel_b, sel_at, sel_bt, out_p):
    dinv = _dinv_packed(degp)
    s2_p = dinv * (acc[0, :NP] + acc[1, :NP] + h2p[...]) + b2p[...]
    t = (jnp.dot(jnp.maximum(s2_p, 0.0), wl2ak[...],
                 preferred_element_type=jnp.float32)
         + jnp.dot(jnp.maximum(xloc[...], 0.0), wl2bk[...],
                   preferred_element_type=jnp.float32)
         + bl2p[...])
    y = jnp.maximum(t, 0.0)
    z = jnp.dot(y, wl3k[...], preferred_element_type=jnp.float32) + bl3p[...]
    za = jnp.dot(z, sel_a[...], preferred_element_type=jnp.float32)
    zb = jnp.dot(z, sel_b[...], preferred_element_type=jnp.float32)
    m = jnp.maximum(za, zb)
    lse = m + jnp.log(jnp.exp(za - m) + jnp.exp(zb - m))
    oa = za - lse
    ob = zb - lse
    out_p[...] = (
        jnp.dot(oa, sel_at[...], preferred_element_type=jnp.float32)
        + jnp.dot(ob, sel_bt[...], preferred_element_type=jnp.float32))


def _full(shape):
    nd = len(shape)
    return pl.BlockSpec(shape, lambda: (0,) * nd)


_pre_a_call = pl.pallas_call(
    _pre_a_body,
    in_specs=[_full((NP, 1024)), _full((1024, 128)), _full((NP, 752)),
              _full((752, 800)), _full((1, 800))],
    out_specs=[_full((NP, 128)), _full((NP, 800))],
    out_shape=[jax.ShapeDtypeStruct((NP, 128), jnp.float32),
               jax.ShapeDtypeStruct((NP, 800), jnp.float32)],
)

_pre_b_call = pl.pallas_call(
    _pre_b_body,
    in_specs=[_full((NC, NP_ACC, 128)), _full((NP, 128))],
    out_specs=[_full((NP, 128))],
    out_shape=[jax.ShapeDtypeStruct((NP, 128), jnp.float32)],
)

_mid_call = pl.pallas_call(
    _mid_body,
    in_specs=[_full((NC, NP_ACC, 128)), _full((NC, NP_ACC, 128)),
              _full((NP, 128)), _full((1, 128)), _full((128, 128))],
    out_specs=[_full((NP, 128))],
    out_shape=[jax.ShapeDtypeStruct((NP, 128), jnp.float32)],
)

_post_call = pl.pallas_call(
    _post_body,
    in_specs=[_full((NC, NP_ACC, 128)), _full((NC, NP_ACC, 128)),
              _full((NP, 128)), _full((1, 128)), _full((NP, 800)),
              _full((128, 648)), _full((800, 648)), _full((1, 648)),
              _full((648, 16)), _full((1, 16)), _full((16, 8)),
              _full((16, 8)), _full((8, 16)), _full((8, 16))],
    out_specs=[_full((NP, 16))],
    out_shape=[jax.ShapeDtypeStruct((NP, 16), jnp.float32)],
)


def kernel(x_graph, x_linear, edge_index, W1, b1, W2, b2,
           Wl1, bl1, Wl2, bl2, Wl3, bl3):
    f32 = jnp.float32
    ei = edge_index.astype(jnp.int32)
    pad_n = E_PAD - E
    pad_src = jnp.arange(pad_n, dtype=jnp.int32) * 131 % N
    src2d = jnp.concatenate([ei[0], pad_src]).reshape(NCHUNKS, B)
    pad_dst = N + jnp.arange(pad_n, dtype=jnp.int32) % (N_ACC - N)
    dst2d = jnp.concatenate([ei[1], pad_dst]).reshape(NCHUNKS, B)
    zeros = jnp.zeros((N_ACC, H1), f32)
    ones_blk = jnp.ones((B, H1), f32)

    eye8 = jnp.eye(8, dtype=f32)
    w2p = jnp.pad(W2, ((0, 0), (0, H1 - H2)))
    wl2a = jnp.pad(Wl2[:H2], ((0, H1 - H2), (0, 0)))
    wl2b = Wl2[H2:]
    w1k = jnp.kron(eye8, W1)
    wl1k = jnp.kron(eye8, Wl1)
    w2k = jnp.kron(eye8, w2p)
    wl2ak = jnp.kron(eye8, wl2a)
    wl2bk = jnp.kron(eye8, wl2b)
    wl3k = jnp.kron(eye8, Wl3)
    sel_a = jnp.kron(eye8, jnp.array([[1.0], [0.0]], f32))
    sel_b = jnp.kron(eye8, jnp.array([[0.0], [1.0]], f32))
    b1p = jnp.tile(b1, 8).reshape(1, 128)
    b2p = jnp.tile(jnp.pad(b2, (0, H1 - H2)), 8).reshape(1, 128)
    bl1p = jnp.tile(bl1, 8).reshape(1, 800)
    bl2p = jnp.tile(bl2, 8).reshape(1, 648)
    bl3p = jnp.tile(bl3, 8).reshape(1, 16)

    xgp = x_graph.reshape(NP, 1024)
    xlp = x_linear.reshape(NP, 752)

    h1r_p, xloc_p = _pre_a_call(xgp, w1k, xlp, wl1k, bl1p)
    degp = _deg_kernel(dst2d, ones_blk, zeros)
    degp_p = degp.reshape(NC, NP_ACC, 128)
    (h1p_p,) = _pre_b_call(degp_p, h1r_p)
    acc1 = _segsum_kernel(h1p_p.reshape(N, H1), src2d, dst2d, zeros)
    (h2p_p,) = _mid_call(degp_p, acc1.reshape(NC, NP_ACC, 128), h1p_p,
                         b1p, w2k)
    acc2 = _segsum_kernel(h2p_p.reshape(N, H1), src2d, dst2d, zeros)
    (out_p,) = _post_call(degp_p, acc2.reshape(NC, NP_ACC, 128), h2p_p,
                          b2p, xloc_p, wl2ak, wl2bk, bl2p, wl3k, bl3p,
                          sel_a, sel_b, sel_a.T, sel_b.T)
    return out_p.reshape(N, 2)
